# baseline (speedup 1.0000x reference)
import jax
import jax.numpy as jnp
from jax import lax
from jax.experimental import pallas as pl
from jax.experimental.pallas import tpu as pltpu
from jax.experimental.pallas import tpu_sc as plsc

VOCAB = 1_000_000
D = 16
F = 26
L = 20
B = 4096
TASK_DIM = 128
PER_ROW = F * L
OUT_D = F * D + TASK_DIM

NC, NS = 2, 16
NW = NC * NS
B_W = B // NW
C = 4
N_CHUNK = B_W // C
N_PAIR = N_CHUNK // 2
IDX_CHUNK = C * PER_ROW


TCH = 1024
N_TCH = VOCAB // TCH
T_TAIL = 512
N_TAIL64 = 64


def _transpose_body(table_t_hbm, tail_hbm, out_hbm, buf_in, buf_out, semi):
    wid = lax.axis_index("s") * NC + lax.axis_index("c")
    iota16 = lax.iota(jnp.int32, D)

    def do_chunk(v0, n):
        pltpu.async_copy(table_t_hbm.at[pl.ds(0, 8), pl.ds(v0, n)],
                         buf_in.at[pl.ds(0, 8), pl.ds(0, n)], semi)
        pltpu.async_copy(table_t_hbm.at[pl.ds(8, 8), pl.ds(v0, n)],
                         buf_in.at[pl.ds(8, 8), pl.ds(0, n)], semi)
        pltpu.make_async_copy(table_t_hbm.at[pl.ds(0, 8), pl.ds(v0, n)],
                              buf_in.at[pl.ds(0, 8), pl.ds(0, n)], semi).wait()
        pltpu.make_async_copy(table_t_hbm.at[pl.ds(8, 8), pl.ds(v0, n)],
                              buf_in.at[pl.ds(8, 8), pl.ds(0, n)], semi).wait()

        def row_body(r8, carry):
            for u in range(8):
                v = r8 * 8 + u
                col = plsc.load_gather(buf_in, [iota16, jnp.full((D,), v,
                                                                 jnp.int32)])
                buf_out[pl.ds(v * D, D)] = col
            return carry

        lax.fori_loop(0, n // 8, row_body, 0)
        pltpu.sync_copy(buf_out.at[pl.ds(0, n * D)],
                        out_hbm.at[pl.ds(v0 * D, n * D)])

    n_mine = (N_TCH - wid + NW - 1) // NW

    def chunk_loop(it, carry):
        do_chunk((wid + it * NW) * TCH, TCH)
        return carry

    lax.fori_loop(0, n_mine, chunk_loop, 0)

    @pl.when(wid == NW - 2)
    def _tail512():
        do_chunk(N_TCH * TCH, T_TAIL)

    @pl.when(wid == NW - 1)
    def _tail64():
        pltpu.sync_copy(tail_hbm, buf_out.at[pl.ds(0, N_TAIL64 * D)])
        pltpu.sync_copy(buf_out.at[pl.ds(0, N_TAIL64 * D)],
                        out_hbm.at[pl.ds((VOCAB - N_TAIL64) * D,
                                         N_TAIL64 * D)])


def _gather_body(indices_hbm, task_ids_hbm, table_hbm, task_table_hbm,
                 out_hbm, idx0, idx1, rows0, rows1, out_v, tids_v,
                 task_rows_v, sem0, sem1, semt):
    wid = lax.axis_index("s") * NC + lax.axis_index("c")
    woff_rows = wid * B_W
    woff_idx = woff_rows * PER_ROW

    pltpu.sync_copy(task_ids_hbm.at[pl.ds(woff_rows, B_W)], tids_v)
    pltpu.async_copy(task_table_hbm.at[tids_v], task_rows_v, semt).wait()

    def reduce_chunk(g, rows_v):
        for c in range(C):
            def field_body(f, carry):
                base = c * PER_ROW + f * L
                acc = rows_v[base]
                for l in range(1, L):
                    acc = acc + rows_v[base + l]
                out_v[pl.ds(c * OUT_D + f * D, D)] = acc
                return carry
            lax.fori_loop(0, F, field_body, 0)
            trow = g * C + c
            for r in range(TASK_DIM // 16):
                out_v[pl.ds(c * OUT_D + F * D + r * 16, 16)] = \
                    task_rows_v[trow, pl.ds(r * 16, 16)]
        row_base = woff_rows + g * C
        pltpu.sync_copy(out_v, out_hbm.at[pl.ds(row_base * OUT_D, C * OUT_D)])

    pltpu.sync_copy(indices_hbm.at[pl.ds(woff_idx, IDX_CHUNK)], idx0)
    pltpu.async_copy(table_hbm.at[idx0], rows0, sem0)

    def pair_body(k, carry):
        g0 = 2 * k
        pltpu.sync_copy(
            indices_hbm.at[pl.ds(woff_idx + (g0 + 1) * IDX_CHUNK, IDX_CHUNK)],
            idx1)
        pltpu.async_copy(table_hbm.at[idx1], rows1, sem1)
        pltpu.make_async_copy(table_hbm.at[idx0], rows0, sem0).wait()
        reduce_chunk(g0, rows0)

        @pl.when(k < N_PAIR - 1)
        def _():
            pltpu.sync_copy(
                indices_hbm.at[
                    pl.ds(woff_idx + (g0 + 2) * IDX_CHUNK, IDX_CHUNK)], idx0)
            pltpu.async_copy(table_hbm.at[idx0], rows0, sem0)
        pltpu.make_async_copy(table_hbm.at[idx1], rows1, sem1).wait()
        reduce_chunk(g0 + 1, rows1)
        return carry

    lax.fori_loop(0, N_PAIR, pair_body, 0)


def kernel(indices, task_ids, main_table, task_table):
    idx_flat = indices.reshape(-1)
    mesh = plsc.VectorSubcoreMesh(core_axis_name="c", subcore_axis_name="s")

    transpose = pl.kernel(
        _transpose_body,
        mesh=mesh,
        compiler_params=pltpu.CompilerParams(use_tc_tiling_on_sc=True,
                                             needs_layout_passes=False),
        out_type=jax.ShapeDtypeStruct((VOCAB * D,), jnp.float32),
        scratch_types=[
            pltpu.VMEM((D, TCH), jnp.float32),
            pltpu.VMEM((TCH * D,), jnp.float32),
            pltpu.SemaphoreType.DMA,
        ],
    )
    tail_flat = main_table[VOCAB - N_TAIL64:, :].reshape(-1)
    table_rm = transpose(main_table.T, tail_flat).reshape(VOCAB, D)

    gather = pl.kernel(
        _gather_body,
        mesh=mesh,
        compiler_params=pltpu.CompilerParams(use_tc_tiling_on_sc=False),
        out_type=jax.ShapeDtypeStruct((B * OUT_D,), jnp.float32),
        scratch_types=[
            pltpu.VMEM((IDX_CHUNK,), jnp.int32),
            pltpu.VMEM((IDX_CHUNK,), jnp.int32),
            pltpu.VMEM((IDX_CHUNK, D), jnp.float32),
            pltpu.VMEM((IDX_CHUNK, D), jnp.float32),
            pltpu.VMEM((C * OUT_D,), jnp.float32),
            pltpu.VMEM((B_W,), jnp.int32),
            pltpu.VMEM((B_W, TASK_DIM), jnp.float32),
            pltpu.SemaphoreType.DMA,
            pltpu.SemaphoreType.DMA,
            pltpu.SemaphoreType.DMA,
        ],
    )
    return gather(idx_flat, task_ids, table_rm,
                  task_table).reshape(B, OUT_D)


# speedup vs baseline: 1.8756x; 1.8756x over previous
import jax
import jax.numpy as jnp
from jax import lax
from jax.experimental import pallas as pl
from jax.experimental.pallas import tpu as pltpu
from jax.experimental.pallas import tpu_sc as plsc

VOCAB = 1_000_000
D = 16
F = 26
L = 20
B = 4096
TASK_DIM = 128
PER_ROW = F * L
OUT_D = F * D + TASK_DIM

NC, NS = 2, 16
NW = NC * NS
B_W = B // NW
C = 4
N_CHUNK = B_W // C
N_PAIR = N_CHUNK // 2
IDX_CHUNK = C * PER_ROW


TCH = 1024
N_TCH = VOCAB // TCH
CH_W = 30
N_EXTRA = N_TCH - NW * CH_W
T_TAIL = 512
N_TAIL64 = 64


def _transpose_body(table_t_hbm, tail_hbm, out_hbm, in0, in1, out0, out1,
                    sem0, sem1):
    wid = lax.axis_index("s") * NC + lax.axis_index("c")
    iota16 = lax.iota(jnp.int32, D)

    def start_in(v0, n, buf, sem):
        pltpu.async_copy(table_t_hbm.at[pl.ds(0, 8), pl.ds(v0, n)],
                         buf.at[pl.ds(0, 8), pl.ds(0, n)], sem)
        pltpu.async_copy(table_t_hbm.at[pl.ds(8, 8), pl.ds(v0, n)],
                         buf.at[pl.ds(8, 8), pl.ds(0, n)], sem)

    def wait_in(v0, n, buf, sem):
        pltpu.make_async_copy(table_t_hbm.at[pl.ds(0, 8), pl.ds(v0, n)],
                              buf.at[pl.ds(0, 8), pl.ds(0, n)], sem).wait()
        pltpu.make_async_copy(table_t_hbm.at[pl.ds(8, 8), pl.ds(v0, n)],
                              buf.at[pl.ds(8, 8), pl.ds(0, n)], sem).wait()

    def transpose_out(v0, n, buf, obuf):
        def g_body(g, carry):
            for d in range(D):
                col = buf[d, pl.ds(g * D, D)]
                plsc.store_scatter(obuf, [iota16 * D + (g * D * D + d)], col)
            return carry

        lax.fori_loop(0, n // D, g_body, 0)
        pltpu.sync_copy(obuf.at[pl.ds(0, n * D)],
                        out_hbm.at[pl.ds(v0 * D, n * D)])

    def v_of(c):
        return c * TCH

    base_c = wid * CH_W
    start_in(v_of(base_c), TCH, in0, sem0)

    def pair_body(k, carry):
        c0 = base_c + 2 * k
        start_in(v_of(c0 + 1), TCH, in1, sem1)
        wait_in(v_of(c0), TCH, in0, sem0)
        transpose_out(v_of(c0), TCH, in0, out0)

        @pl.when(k < CH_W // 2 - 1)
        def _():
            start_in(v_of(c0 + 2), TCH, in0, sem0)
        wait_in(v_of(c0 + 1), TCH, in1, sem1)
        transpose_out(v_of(c0 + 1), TCH, in1, out1)
        return carry

    lax.fori_loop(0, CH_W // 2, pair_body, 0)

    @pl.when(wid < N_EXTRA)
    def _extra():
        v0 = v_of(NW * CH_W) + wid * TCH
        start_in(v0, TCH, in0, sem0)
        wait_in(v0, TCH, in0, sem0)
        transpose_out(v0, TCH, in0, out0)

    @pl.when(wid == N_EXTRA)
    def _tail512():
        v0 = N_TCH * TCH
        start_in(v0, T_TAIL, in0, sem0)
        wait_in(v0, T_TAIL, in0, sem0)
        transpose_out(v0, T_TAIL, in0, out0)

    @pl.when(wid == N_EXTRA + 1)
    def _tail64():
        pltpu.sync_copy(tail_hbm, out0.at[pl.ds(0, N_TAIL64 * D)])
        pltpu.sync_copy(out0.at[pl.ds(0, N_TAIL64 * D)],
                        out_hbm.at[pl.ds((VOCAB - N_TAIL64) * D,
                                         N_TAIL64 * D)])


def _gather_body(indices_hbm, task_ids_hbm, table_hbm, task_table_hbm,
                 out_hbm, idx0, idx1, rows0, rows1, out_v, tids_v,
                 task_rows_v, sem0, sem1, semt):
    wid = lax.axis_index("s") * NC + lax.axis_index("c")
    woff_rows = wid * B_W
    woff_idx = woff_rows * PER_ROW

    pltpu.sync_copy(task_ids_hbm.at[pl.ds(woff_rows, B_W)], tids_v)
    pltpu.async_copy(task_table_hbm.at[tids_v], task_rows_v, semt).wait()

    def reduce_chunk(g, rows_v):
        for c in range(C):
            def field_body(f, carry):
                base = c * PER_ROW + f * L
                acc = rows_v[base]
                for l in range(1, L):
                    acc = acc + rows_v[base + l]
                out_v[pl.ds(c * OUT_D + f * D, D)] = acc
                return carry
            lax.fori_loop(0, F, field_body, 0)
            trow = g * C + c
            for r in range(TASK_DIM // 16):
                out_v[pl.ds(c * OUT_D + F * D + r * 16, 16)] = \
                    task_rows_v[trow, pl.ds(r * 16, 16)]
        row_base = woff_rows + g * C
        pltpu.sync_copy(out_v, out_hbm.at[pl.ds(row_base * OUT_D, C * OUT_D)])

    pltpu.sync_copy(indices_hbm.at[pl.ds(woff_idx, IDX_CHUNK)], idx0)
    pltpu.async_copy(table_hbm.at[idx0], rows0, sem0)

    def pair_body(k, carry):
        g0 = 2 * k
        pltpu.sync_copy(
            indices_hbm.at[pl.ds(woff_idx + (g0 + 1) * IDX_CHUNK, IDX_CHUNK)],
            idx1)
        pltpu.async_copy(table_hbm.at[idx1], rows1, sem1)
        pltpu.make_async_copy(table_hbm.at[idx0], rows0, sem0).wait()
        reduce_chunk(g0, rows0)

        @pl.when(k < N_PAIR - 1)
        def _():
            pltpu.sync_copy(
                indices_hbm.at[
                    pl.ds(woff_idx + (g0 + 2) * IDX_CHUNK, IDX_CHUNK)], idx0)
            pltpu.async_copy(table_hbm.at[idx0], rows0, sem0)
        pltpu.make_async_copy(table_hbm.at[idx1], rows1, sem1).wait()
        reduce_chunk(g0 + 1, rows1)
        return carry

    lax.fori_loop(0, N_PAIR, pair_body, 0)


def kernel(indices, task_ids, main_table, task_table):
    idx_flat = indices.reshape(-1)
    mesh = plsc.VectorSubcoreMesh(core_axis_name="c", subcore_axis_name="s")

    transpose = pl.kernel(
        _transpose_body,
        mesh=mesh,
        compiler_params=pltpu.CompilerParams(use_tc_tiling_on_sc=True,
                                             needs_layout_passes=False),
        out_type=jax.ShapeDtypeStruct((VOCAB * D,), jnp.float32),
        scratch_types=[
            pltpu.VMEM((D, TCH), jnp.float32),
            pltpu.VMEM((D, TCH), jnp.float32),
            pltpu.VMEM((TCH * D,), jnp.float32),
            pltpu.VMEM((TCH * D,), jnp.float32),
            pltpu.SemaphoreType.DMA,
            pltpu.SemaphoreType.DMA,
        ],
    )
    tail_flat = main_table[VOCAB - N_TAIL64:, :].reshape(-1)
    table_rm = transpose(main_table.T, tail_flat).reshape(VOCAB, D)

    gather = pl.kernel(
        _gather_body,
        mesh=mesh,
        compiler_params=pltpu.CompilerParams(use_tc_tiling_on_sc=False),
        out_type=jax.ShapeDtypeStruct((B * OUT_D,), jnp.float32),
        scratch_types=[
            pltpu.VMEM((IDX_CHUNK,), jnp.int32),
            pltpu.VMEM((IDX_CHUNK,), jnp.int32),
            pltpu.VMEM((IDX_CHUNK, D), jnp.float32),
            pltpu.VMEM((IDX_CHUNK, D), jnp.float32),
            pltpu.VMEM((C * OUT_D,), jnp.float32),
            pltpu.VMEM((B_W,), jnp.int32),
            pltpu.VMEM((B_W, TASK_DIM), jnp.float32),
            pltpu.SemaphoreType.DMA,
            pltpu.SemaphoreType.DMA,
            pltpu.SemaphoreType.DMA,
        ],
    )
    return gather(idx_flat, task_ids, table_rm,
                  task_table).reshape(B, OUT_D)


# speedup vs baseline: 1.9832x; 1.0574x over previous
import jax
import jax.numpy as jnp
from jax import lax
from jax.experimental import pallas as pl
from jax.experimental.pallas import tpu as pltpu
from jax.experimental.pallas import tpu_sc as plsc

VOCAB = 1_000_000
D = 16
F = 26
L = 20
B = 4096
TASK_DIM = 128
PER_ROW = F * L
OUT_D = F * D + TASK_DIM

NC, NS = 2, 16
NW = NC * NS
B_W = B // NW
C = 4
N_CHUNK = B_W // C
N_PAIR = N_CHUNK // 2
IDX_CHUNK = C * PER_ROW


TCH = 1024
N_TCH = VOCAB // TCH
CH_W = 30
N_EXTRA = N_TCH - NW * CH_W
T_TAIL = 512
N_TAIL64 = 64


def _transpose_body(table_t_hbm, tail_hbm, out_hbm, in0, in1, out0, out1,
                    sem0, sem1):
    wid = lax.axis_index("s") * NC + lax.axis_index("c")
    iota16 = lax.iota(jnp.int32, D)

    def start_in(v0, n, buf, sem):
        pltpu.async_copy(table_t_hbm.at[pl.ds(0, 8), pl.ds(v0, n)],
                         buf.at[pl.ds(0, 8), pl.ds(0, n)], sem)
        pltpu.async_copy(table_t_hbm.at[pl.ds(8, 8), pl.ds(v0, n)],
                         buf.at[pl.ds(8, 8), pl.ds(0, n)], sem)

    def wait_in(v0, n, buf, sem):
        pltpu.make_async_copy(table_t_hbm.at[pl.ds(0, 8), pl.ds(v0, n)],
                              buf.at[pl.ds(0, 8), pl.ds(0, n)], sem).wait()
        pltpu.make_async_copy(table_t_hbm.at[pl.ds(8, 8), pl.ds(v0, n)],
                              buf.at[pl.ds(8, 8), pl.ds(0, n)], sem).wait()

    def transpose_out(v0, n, buf, obuf):
        def g_body(g, carry):
            for d in range(D):
                col = buf[d, pl.ds(g * D, D)]
                plsc.store_scatter(obuf, [iota16 * D + (g * D * D + d)], col)
            return carry

        lax.fori_loop(0, n // D, g_body, 0)
        pltpu.sync_copy(obuf.at[pl.ds(0, n * D)],
                        out_hbm.at[pl.ds(v0 * D, n * D)])

    def v_of(c):
        return c * TCH

    base_c = wid * CH_W
    start_in(v_of(base_c), TCH, in0, sem0)

    def pair_body(k, carry):
        c0 = base_c + 2 * k
        start_in(v_of(c0 + 1), TCH, in1, sem1)
        wait_in(v_of(c0), TCH, in0, sem0)
        transpose_out(v_of(c0), TCH, in0, out0)

        @pl.when(k < CH_W // 2 - 1)
        def _():
            start_in(v_of(c0 + 2), TCH, in0, sem0)
        wait_in(v_of(c0 + 1), TCH, in1, sem1)
        transpose_out(v_of(c0 + 1), TCH, in1, out1)
        return carry

    lax.fori_loop(0, CH_W // 2, pair_body, 0)

    @pl.when(wid < N_EXTRA)
    def _extra():
        v0 = v_of(NW * CH_W) + wid * TCH
        start_in(v0, TCH, in0, sem0)
        wait_in(v0, TCH, in0, sem0)
        transpose_out(v0, TCH, in0, out0)

    @pl.when(wid == N_EXTRA)
    def _tail512():
        v0 = N_TCH * TCH
        start_in(v0, T_TAIL, in0, sem0)
        wait_in(v0, T_TAIL, in0, sem0)
        transpose_out(v0, T_TAIL, in0, out0)

    @pl.when(wid == N_EXTRA + 1)
    def _tail64():
        pltpu.sync_copy(tail_hbm, out0.at[pl.ds(0, N_TAIL64 * D)])
        pltpu.sync_copy(out0.at[pl.ds(0, N_TAIL64 * D)],
                        out_hbm.at[pl.ds((VOCAB - N_TAIL64) * D,
                                         N_TAIL64 * D)])


def _gather_body(indices_hbm, task_ids_hbm, table_hbm, task_table_hbm,
                 out_hbm, idx0, idx1, rows0, rows1, out_v, tids_v,
                 task_rows_v, sem0, sem1, semi0, semi1, semt):
    wid = lax.axis_index("s") * NC + lax.axis_index("c")
    woff_rows = wid * B_W
    woff_idx = woff_rows * PER_ROW

    def start_idx(g, idx_v, semi):
        pltpu.async_copy(
            indices_hbm.at[pl.ds(woff_idx + g * IDX_CHUNK, IDX_CHUNK)],
            idx_v, semi)

    def wait_idx(g, idx_v, semi):
        pltpu.make_async_copy(
            indices_hbm.at[pl.ds(woff_idx + g * IDX_CHUNK, IDX_CHUNK)],
            idx_v, semi).wait()

    pltpu.sync_copy(task_ids_hbm.at[pl.ds(woff_rows, B_W)], tids_v)
    pltpu.async_copy(task_table_hbm.at[tids_v], task_rows_v, semt).wait()

    def reduce_chunk(g, rows_v):
        for c in range(C):
            def field_body(f, carry):
                base = c * PER_ROW + f * L
                acc = rows_v[base]
                for l in range(1, L):
                    acc = acc + rows_v[base + l]
                out_v[pl.ds(c * OUT_D + f * D, D)] = acc
                return carry
            lax.fori_loop(0, F, field_body, 0)
            trow = g * C + c
            for r in range(TASK_DIM // 16):
                out_v[pl.ds(c * OUT_D + F * D + r * 16, 16)] = \
                    task_rows_v[trow, pl.ds(r * 16, 16)]
        row_base = woff_rows + g * C
        pltpu.sync_copy(out_v, out_hbm.at[pl.ds(row_base * OUT_D, C * OUT_D)])

    start_idx(0, idx0, semi0)
    wait_idx(0, idx0, semi0)
    pltpu.async_copy(table_hbm.at[idx0], rows0, sem0)
    start_idx(1, idx1, semi1)

    def pair_body(k, carry):
        g0 = 2 * k
        wait_idx(g0 + 1, idx1, semi1)
        pltpu.async_copy(table_hbm.at[idx1], rows1, sem1)
        pltpu.make_async_copy(table_hbm.at[idx0], rows0, sem0).wait()

        @pl.when(k < N_PAIR - 1)
        def _():
            start_idx(g0 + 2, idx0, semi0)
        reduce_chunk(g0, rows0)

        @pl.when(k < N_PAIR - 1)
        def _():
            wait_idx(g0 + 2, idx0, semi0)
            pltpu.async_copy(table_hbm.at[idx0], rows0, sem0)
            start_idx(g0 + 3, idx1, semi1)
        pltpu.make_async_copy(table_hbm.at[idx1], rows1, sem1).wait()
        reduce_chunk(g0 + 1, rows1)
        return carry

    lax.fori_loop(0, N_PAIR, pair_body, 0)


def kernel(indices, task_ids, main_table, task_table):
    idx_flat = indices.reshape(-1)
    mesh = plsc.VectorSubcoreMesh(core_axis_name="c", subcore_axis_name="s")

    transpose = pl.kernel(
        _transpose_body,
        mesh=mesh,
        compiler_params=pltpu.CompilerParams(use_tc_tiling_on_sc=True,
                                             needs_layout_passes=False),
        out_type=jax.ShapeDtypeStruct((VOCAB * D,), jnp.float32),
        scratch_types=[
            pltpu.VMEM((D, TCH), jnp.float32),
            pltpu.VMEM((D, TCH), jnp.float32),
            pltpu.VMEM((TCH * D,), jnp.float32),
            pltpu.VMEM((TCH * D,), jnp.float32),
            pltpu.SemaphoreType.DMA,
            pltpu.SemaphoreType.DMA,
        ],
    )
    tail_flat = main_table[VOCAB - N_TAIL64:, :].reshape(-1)
    table_rm = transpose(main_table.T, tail_flat).reshape(VOCAB, D)

    gather = pl.kernel(
        _gather_body,
        mesh=mesh,
        compiler_params=pltpu.CompilerParams(use_tc_tiling_on_sc=False),
        out_type=jax.ShapeDtypeStruct((B * OUT_D,), jnp.float32),
        scratch_types=[
            pltpu.VMEM((IDX_CHUNK,), jnp.int32),
            pltpu.VMEM((IDX_CHUNK,), jnp.int32),
            pltpu.VMEM((IDX_CHUNK, D), jnp.float32),
            pltpu.VMEM((IDX_CHUNK, D), jnp.float32),
            pltpu.VMEM((C * OUT_D,), jnp.float32),
            pltpu.VMEM((B_W,), jnp.int32),
            pltpu.VMEM((B_W, TASK_DIM), jnp.float32),
            pltpu.SemaphoreType.DMA,
            pltpu.SemaphoreType.DMA,
            pltpu.SemaphoreType.DMA,
            pltpu.SemaphoreType.DMA,
            pltpu.SemaphoreType.DMA,
        ],
    )
    return gather(idx_flat, task_ids, table_rm,
                  task_table).reshape(B, OUT_D)


# speedup vs baseline: 2.0980x; 1.0579x over previous
import jax
import jax.numpy as jnp
from jax import lax
from jax.experimental import pallas as pl
from jax.experimental.pallas import tpu as pltpu
from jax.experimental.pallas import tpu_sc as plsc

VOCAB = 1_000_000
D = 16
F = 26
L = 20
B = 4096
TASK_DIM = 128
PER_ROW = F * L
OUT_D = F * D + TASK_DIM

NC, NS = 2, 16
NW = NC * NS
B_W = B // NW
C = 4
N_CHUNK = B_W // C
N_PAIR = N_CHUNK // 2
IDX_CHUNK = C * PER_ROW


TCH = 1024
N_TCH = VOCAB // TCH
CH_W = 30
N_EXTRA = N_TCH - NW * CH_W
T_TAIL = 512
N_TAIL64 = 64


def _transpose_body(table_t_hbm, tail_hbm, out_hbm, in0, in1, out0, out1,
                    sem0, sem1, semo0, semo1):
    wid = lax.axis_index("s") * NC + lax.axis_index("c")
    iota16 = lax.iota(jnp.int32, D)

    def start_in(v0, n, buf, sem):
        pltpu.async_copy(table_t_hbm.at[pl.ds(0, 8), pl.ds(v0, n)],
                         buf.at[pl.ds(0, 8), pl.ds(0, n)], sem)
        pltpu.async_copy(table_t_hbm.at[pl.ds(8, 8), pl.ds(v0, n)],
                         buf.at[pl.ds(8, 8), pl.ds(0, n)], sem)

    def wait_in(v0, n, buf, sem):
        pltpu.make_async_copy(table_t_hbm.at[pl.ds(0, 8), pl.ds(v0, n)],
                              buf.at[pl.ds(0, 8), pl.ds(0, n)], sem).wait()
        pltpu.make_async_copy(table_t_hbm.at[pl.ds(8, 8), pl.ds(v0, n)],
                              buf.at[pl.ds(8, 8), pl.ds(0, n)], sem).wait()

    def transpose_fill(n, buf, obuf):
        def g_body(g, carry):
            for d in range(D):
                col = buf[d, pl.ds(g * D, D)]
                plsc.store_scatter(obuf, [iota16 * D + (g * D * D + d)], col)
            return carry

        lax.fori_loop(0, n // D, g_body, 0)

    def out_start(v0, n, obuf, semo):
        pltpu.async_copy(obuf.at[pl.ds(0, n * D)],
                         out_hbm.at[pl.ds(v0 * D, n * D)], semo)

    def out_wait(n, obuf, semo):
        pltpu.make_async_copy(obuf.at[pl.ds(0, n * D)],
                              out_hbm.at[pl.ds(0, n * D)], semo).wait()

    def transpose_out(v0, n, buf, obuf):
        transpose_fill(n, buf, obuf)
        pltpu.sync_copy(obuf.at[pl.ds(0, n * D)],
                        out_hbm.at[pl.ds(v0 * D, n * D)])

    def v_of(c):
        return c * TCH

    base_c = wid * CH_W
    start_in(v_of(base_c), TCH, in0, sem0)

    def pair_body(k, carry):
        c0 = base_c + 2 * k
        start_in(v_of(c0 + 1), TCH, in1, sem1)
        wait_in(v_of(c0), TCH, in0, sem0)

        @pl.when(k > 0)
        def _():
            out_wait(TCH, out0, semo0)
        transpose_fill(TCH, in0, out0)
        out_start(v_of(c0), TCH, out0, semo0)

        @pl.when(k < CH_W // 2 - 1)
        def _():
            start_in(v_of(c0 + 2), TCH, in0, sem0)
        wait_in(v_of(c0 + 1), TCH, in1, sem1)

        @pl.when(k > 0)
        def _():
            out_wait(TCH, out1, semo1)
        transpose_fill(TCH, in1, out1)
        out_start(v_of(c0 + 1), TCH, out1, semo1)
        return carry

    lax.fori_loop(0, CH_W // 2, pair_body, 0)
    out_wait(TCH, out0, semo0)
    out_wait(TCH, out1, semo1)

    @pl.when(wid < N_EXTRA)
    def _extra():
        v0 = v_of(NW * CH_W) + wid * TCH
        start_in(v0, TCH, in0, sem0)
        wait_in(v0, TCH, in0, sem0)
        transpose_out(v0, TCH, in0, out0)

    @pl.when(wid == N_EXTRA)
    def _tail512():
        v0 = N_TCH * TCH
        start_in(v0, T_TAIL, in0, sem0)
        wait_in(v0, T_TAIL, in0, sem0)
        transpose_out(v0, T_TAIL, in0, out0)

    @pl.when(wid == N_EXTRA + 1)
    def _tail64():
        pltpu.sync_copy(tail_hbm, out0.at[pl.ds(0, N_TAIL64 * D)])
        pltpu.sync_copy(out0.at[pl.ds(0, N_TAIL64 * D)],
                        out_hbm.at[pl.ds((VOCAB - N_TAIL64) * D,
                                         N_TAIL64 * D)])


def _gather_body(indices_hbm, task_ids_hbm, table_hbm, task_table_hbm,
                 out_hbm, idx0, idx1, rows0, rows1, out_v, tids_v,
                 task_rows_v, sem0, sem1, semi0, semi1, semt):
    wid = lax.axis_index("s") * NC + lax.axis_index("c")
    woff_rows = wid * B_W
    woff_idx = woff_rows * PER_ROW

    def start_idx(g, idx_v, semi):
        pltpu.async_copy(
            indices_hbm.at[pl.ds(woff_idx + g * IDX_CHUNK, IDX_CHUNK)],
            idx_v, semi)

    def wait_idx(g, idx_v, semi):
        pltpu.make_async_copy(
            indices_hbm.at[pl.ds(woff_idx + g * IDX_CHUNK, IDX_CHUNK)],
            idx_v, semi).wait()

    pltpu.sync_copy(task_ids_hbm.at[pl.ds(woff_rows, B_W)], tids_v)
    pltpu.async_copy(task_table_hbm.at[tids_v], task_rows_v, semt).wait()

    def reduce_chunk(g, rows_v):
        for c in range(C):
            def field_body(f, carry):
                base = c * PER_ROW + f * L
                acc = rows_v[base]
                for l in range(1, L):
                    acc = acc + rows_v[base + l]
                out_v[pl.ds(c * OUT_D + f * D, D)] = acc
                return carry
            lax.fori_loop(0, F, field_body, 0)
            trow = g * C + c
            for r in range(TASK_DIM // 16):
                out_v[pl.ds(c * OUT_D + F * D + r * 16, 16)] = \
                    task_rows_v[trow, pl.ds(r * 16, 16)]
        row_base = woff_rows + g * C
        pltpu.sync_copy(out_v, out_hbm.at[pl.ds(row_base * OUT_D, C * OUT_D)])

    start_idx(0, idx0, semi0)
    wait_idx(0, idx0, semi0)
    pltpu.async_copy(table_hbm.at[idx0], rows0, sem0)
    start_idx(1, idx1, semi1)

    def pair_body(k, carry):
        g0 = 2 * k
        wait_idx(g0 + 1, idx1, semi1)
        pltpu.async_copy(table_hbm.at[idx1], rows1, sem1)
        pltpu.make_async_copy(table_hbm.at[idx0], rows0, sem0).wait()

        @pl.when(k < N_PAIR - 1)
        def _():
            start_idx(g0 + 2, idx0, semi0)
        reduce_chunk(g0, rows0)

        @pl.when(k < N_PAIR - 1)
        def _():
            wait_idx(g0 + 2, idx0, semi0)
            pltpu.async_copy(table_hbm.at[idx0], rows0, sem0)
            start_idx(g0 + 3, idx1, semi1)
        pltpu.make_async_copy(table_hbm.at[idx1], rows1, sem1).wait()
        reduce_chunk(g0 + 1, rows1)
        return carry

    lax.fori_loop(0, N_PAIR, pair_body, 0)


def kernel(indices, task_ids, main_table, task_table):
    idx_flat = indices.reshape(-1)
    mesh = plsc.VectorSubcoreMesh(core_axis_name="c", subcore_axis_name="s")

    transpose = pl.kernel(
        _transpose_body,
        mesh=mesh,
        compiler_params=pltpu.CompilerParams(use_tc_tiling_on_sc=True,
                                             needs_layout_passes=False),
        out_type=jax.ShapeDtypeStruct((VOCAB * D,), jnp.float32),
        scratch_types=[
            pltpu.VMEM((D, TCH), jnp.float32),
            pltpu.VMEM((D, TCH), jnp.float32),
            pltpu.VMEM((TCH * D,), jnp.float32),
            pltpu.VMEM((TCH * D,), jnp.float32),
            pltpu.SemaphoreType.DMA,
            pltpu.SemaphoreType.DMA,
            pltpu.SemaphoreType.DMA,
            pltpu.SemaphoreType.DMA,
        ],
    )
    tail_flat = main_table[VOCAB - N_TAIL64:, :].reshape(-1)
    table_rm = transpose(main_table.T, tail_flat).reshape(VOCAB, D)

    gather = pl.kernel(
        _gather_body,
        mesh=mesh,
        compiler_params=pltpu.CompilerParams(use_tc_tiling_on_sc=False),
        out_type=jax.ShapeDtypeStruct((B * OUT_D,), jnp.float32),
        scratch_types=[
            pltpu.VMEM((IDX_CHUNK,), jnp.int32),
            pltpu.VMEM((IDX_CHUNK,), jnp.int32),
            pltpu.VMEM((IDX_CHUNK, D), jnp.float32),
            pltpu.VMEM((IDX_CHUNK, D), jnp.float32),
            pltpu.VMEM((C * OUT_D,), jnp.float32),
            pltpu.VMEM((B_W,), jnp.int32),
            pltpu.VMEM((B_W, TASK_DIM), jnp.float32),
            pltpu.SemaphoreType.DMA,
            pltpu.SemaphoreType.DMA,
            pltpu.SemaphoreType.DMA,
            pltpu.SemaphoreType.DMA,
            pltpu.SemaphoreType.DMA,
        ],
    )
    return gather(idx_flat, task_ids, table_rm,
                  task_table).reshape(B, OUT_D)
